# baseline (speedup 1.0000x reference)
import functools

import jax
import jax.numpy as jnp
from jax import lax
from jax.experimental import pallas as pl
from jax.experimental.pallas import tpu as pltpu
from jax.experimental.pallas import tpu_sc as plsc

BATCH = 16384
DIM = 64
CHUNK = 128


def _gather_call(ids_grouped, table_pairs, num_cores, b_per_w):
    n_chunks = b_per_w // CHUNK
    mesh = plsc.VectorSubcoreMesh(core_axis_name="c", subcore_axis_name="s")

    @functools.partial(
        pl.kernel,
        mesh=mesh,
        out_type=jax.ShapeDtypeStruct((BATCH, 2 * DIM), jnp.float32),
        compiler_params=pltpu.CompilerParams(use_tc_tiling_on_sc=False),
        scratch_types=[
            pltpu.VMEM((n_chunks, CHUNK), jnp.int32),
            pltpu.VMEM((b_per_w, 2 * DIM), jnp.float32),
            pltpu.SemaphoreType.DMA,
        ],
    )
    def k(ids_hbm, table_hbm, out_hbm, idx_v, rows_v, sem):
        wid = lax.axis_index("s") * num_cores + lax.axis_index("c")
        base = wid * b_per_w
        pltpu.sync_copy(ids_hbm.at[wid], idx_v)
        copies = []
        for j in range(n_chunks):
            copies.append(
                pltpu.async_copy(
                    table_hbm.at[idx_v.at[j]],
                    rows_v.at[pl.ds(j * CHUNK, CHUNK)],
                    sem,
                )
            )
        for c in copies:
            c.wait()
        pltpu.sync_copy(rows_v, out_hbm.at[pl.ds(base, b_per_w)])

    return k(ids_grouped, table_pairs)


def kernel(style_ids, embeddings):
    info = plsc.get_sparse_core_info()
    n_workers = info.num_cores * info.num_subcores
    b_per_w = BATCH // n_workers
    ids = style_ids.astype(jnp.int32)
    ids_grouped = (ids >> 1).reshape(n_workers, b_per_w // CHUNK, CHUNK)
    table_pairs = embeddings.reshape(NUM_PAIR_ROWS, 2 * DIM)
    out_pairs = _gather_call(ids_grouped, table_pairs, info.num_cores, b_per_w)
    parity = (ids & 1).astype(jnp.bool_)
    return jnp.where(parity[:, None], out_pairs[:, DIM:], out_pairs[:, :DIM])


NUM_PAIR_ROWS = 500000


# speedup vs baseline: 2.0320x; 2.0320x over previous
import functools

import jax
import jax.numpy as jnp
from jax import lax
from jax.experimental import pallas as pl
from jax.experimental.pallas import tpu as pltpu
from jax.experimental.pallas import tpu_sc as plsc

BATCH = 16384
DIM = 64
L = 16
FIRE = 4


def _gather_call(ids, table_t, num_cores, b_per_w):
    mesh = plsc.VectorSubcoreMesh(core_axis_name="c", subcore_axis_name="s")

    @functools.partial(
        pl.kernel,
        mesh=mesh,
        out_type=jax.ShapeDtypeStruct((BATCH, 2 * DIM), jnp.float32),
        compiler_params=pltpu.CompilerParams(needs_layout_passes=False),
        scratch_types=[
            pltpu.VMEM((b_per_w,), jnp.int32),
            pltpu.VMEM((FIRE, DIM, 2 * DIM), jnp.float32),
            pltpu.VMEM((b_per_w, 2 * DIM), jnp.float32),
            pltpu.SemaphoreType.DMA,
        ],
    )
    def k(ids_hbm, table_hbm, out_hbm, ids_v, slab_v, rows_v, sem):
        wid = lax.axis_index("s") * num_cores + lax.axis_index("c")
        base = wid * b_per_w
        pltpu.sync_copy(ids_hbm.at[pl.ds(base, b_per_w)], ids_v)
        iota16 = lax.iota(jnp.int32, L)

        def group(g, carry):
            vec = ids_v[pl.ds(g * L, L)]
            for sub in range(L // FIRE):
                copies = []
                scalars = []
                for f in range(FIRE):
                    t = sub * FIRE + f
                    i_s = jnp.sum(jnp.where(iota16 == t, vec, 0))
                    scalars.append(i_s)
                    q = i_s >> 7
                    copies.append(
                        pltpu.async_copy(
                            table_hbm.at[
                                :, pl.ds(pl.multiple_of(q * 128, 128), 128)
                            ],
                            slab_v.at[f],
                            sem,
                        )
                    )
                for c in copies:
                    c.wait()
                for f in range(FIRE):
                    t = sub * FIRE + f
                    kk = g * L + t
                    l_vec = jnp.full((L,), scalars[f] & 127, jnp.int32)
                    for h in range(DIM // L):
                        j_vec = iota16 + h * L
                        vals = plsc.load_gather(slab_v.at[f], [j_vec, l_vec])
                        rows_v[kk, pl.ds(h * L, L)] = vals
            return carry

        lax.fori_loop(0, b_per_w // L, group, 0)
        pltpu.sync_copy(rows_v, out_hbm.at[pl.ds(base, b_per_w)])

    return k(ids, table_t)


def kernel(style_ids, embeddings):
    info = plsc.get_sparse_core_info()
    n_workers = info.num_cores * info.num_subcores
    b_per_w = BATCH // n_workers
    ids = style_ids.astype(jnp.int32)
    out_wide = _gather_call(ids, embeddings.T, info.num_cores, b_per_w)
    return out_wide[:, :DIM]


# speedup vs baseline: 3.2072x; 1.5783x over previous
import functools

import jax
import jax.numpy as jnp
from jax import lax
from jax.experimental import pallas as pl
from jax.experimental.pallas import tpu as pltpu
from jax.experimental.pallas import tpu_sc as plsc

BATCH = 16384
DIM = 64
L = 16
FIRE = 4


def _gather_call(ids, table_t, num_cores, b_per_w):
    mesh = plsc.VectorSubcoreMesh(core_axis_name="c", subcore_axis_name="s")

    @functools.partial(
        pl.kernel,
        mesh=mesh,
        out_type=jax.ShapeDtypeStruct((DIM, BATCH), jnp.float32),
        compiler_params=pltpu.CompilerParams(needs_layout_passes=False),
        scratch_types=[
            pltpu.VMEM((b_per_w,), jnp.int32),
            pltpu.VMEM((2, FIRE, DIM, 2 * DIM), jnp.float32),
            pltpu.VMEM((DIM, b_per_w), jnp.float32),
            pltpu.SemaphoreType.DMA,
        ],
    )
    def k(ids_hbm, table_hbm, out_hbm, ids_v, slab_v, rows_v, sem):
        wid = lax.axis_index("s") * num_cores + lax.axis_index("c")
        base = wid * b_per_w
        pltpu.sync_copy(ids_hbm.at[pl.ds(base, b_per_w)], ids_v)
        iota16 = lax.iota(jnp.int32, L)
        n_sub = b_per_w // FIRE

        def scalar_id(sg, f):
            vec = ids_v[pl.ds((sg // 4) * L, L)]
            t = (sg % 4) * FIRE + f
            return jnp.sum(jnp.where(iota16 == t, vec, 0))

        def fire(sg, buf):
            for f in range(FIRE):
                q = scalar_id(sg, f) >> 7
                pltpu.async_copy(
                    table_hbm.at[:, pl.ds(pl.multiple_of(q * 128, 128), 128)],
                    slab_v.at[buf, f],
                    sem,
                )

        fire(0, 0)

        def group(sg, carry):
            buf = lax.rem(sg, 2)

            @pl.when(sg + 1 < n_sub)
            def _():
                fire(sg + 1, lax.rem(sg + 1, 2))

            for f in range(FIRE):
                pltpu.make_async_copy(
                    table_hbm.at[:, pl.ds(0, 128)], slab_v.at[buf, f], sem
                ).wait()
            for f in range(FIRE):
                i_s = scalar_id(sg, f)
                kk = sg * FIRE + f
                l_vec = jnp.full((L,), i_s & 127, jnp.int32)
                kk_vec = jnp.full((L,), kk, jnp.int32)
                for h in range(DIM // L):
                    j_vec = iota16 + h * L
                    vals = plsc.load_gather(slab_v.at[buf, f], [j_vec, l_vec])
                    plsc.store_scatter(rows_v, [j_vec, kk_vec], vals)
            return carry

        lax.fori_loop(0, n_sub, group, 0)
        pltpu.sync_copy(rows_v, out_hbm.at[:, pl.ds(base, b_per_w)])

    return k(ids, table_t)


def kernel(style_ids, embeddings):
    info = plsc.get_sparse_core_info()
    n_workers = info.num_cores * info.num_subcores
    b_per_w = BATCH // n_workers
    ids = style_ids.astype(jnp.int32)
    out_t = _gather_call(ids, embeddings.T, info.num_cores, b_per_w)
    return out_t.T
